# Initial kernel scaffold; baseline (speedup 1.0000x reference)
import functools

import jax
import jax.numpy as jnp
from jax import lax
from jax.experimental import pallas as pl
from jax.experimental.pallas import tpu as pltpu
from jax.experimental.pallas import tpu_sc as plsc

VOCAB = 1000000
EMBED = 32
HIDDEN = 256
OUT = 1
BATCH = 16384
HIST = 200

NC = 2
NS = 16
NW = NC * NS
ELEMS = BATCH // NW
CB = 16
NCHUNK = ELEMS // CB
H0 = 96
H1 = HIST - H0


def _pool_body(inputs_hbm, table_hbm, out_hbm, idx_v, rows_v, pool_v, sem):
    c = lax.axis_index("c")
    s = lax.axis_index("s")
    wid = s * NC + c
    base = wid * ELEMS

    def chunk_body(ci, _):
        row0 = base + ci * CB
        pltpu.sync_copy(inputs_hbm.at[pl.ds(row0, CB)], idx_v)

        def elem_body(e, _):
            cp0 = pltpu.async_copy(
                table_hbm.at[idx_v.at[e, pl.ds(0, H0)]],
                rows_v.at[pl.ds(0, H0)], sem)
            cp1 = pltpu.async_copy(
                table_hbm.at[idx_v.at[e, pl.ds(H0, H1)]],
                rows_v.at[pl.ds(H0, H1)], sem)
            cp0.wait()
            cp1.wait()

            def row_body(l, accs):
                a0, a1 = accs
                return (a0 + rows_v[l, pl.ds(0, 16)],
                        a1 + rows_v[l, pl.ds(16, 16)])

            a0, a1 = lax.fori_loop(
                0, HIST, row_body,
                (jnp.zeros((16,), jnp.float32), jnp.zeros((16,), jnp.float32)),
                unroll=4)
            pool_v[e, pl.ds(0, 16)] = a0 * (1.0 / HIST)
            pool_v[e, pl.ds(16, 16)] = a1 * (1.0 / HIST)
            return 0

        lax.fori_loop(0, CB, elem_body, 0)
        pltpu.sync_copy(pool_v, out_hbm.at[pl.ds(row0, CB)])
        return 0

    lax.fori_loop(0, NCHUNK, chunk_body, 0)


def _pool(inputs, emb_table):
    mesh = plsc.VectorSubcoreMesh(core_axis_name="c", subcore_axis_name="s")
    fn = functools.partial(
        pl.kernel,
        mesh=mesh,
        out_type=jax.ShapeDtypeStruct((BATCH, EMBED), jnp.float32),
        scratch_types=[
            pltpu.VMEM((CB, HIST), jnp.int32),
            pltpu.VMEM((HIST, EMBED), jnp.float32),
            pltpu.VMEM((CB, EMBED), jnp.float32),
            pltpu.SemaphoreType.DMA,
        ],
    )(_pool_body)
    return fn(inputs, emb_table)


def _mlp_body(x_ref, w1_ref, b1_ref, w2_ref, b2_ref, o_ref):
    x = x_ref[...]
    h = jnp.dot(x, w1_ref[...], preferred_element_type=jnp.float32)
    h = jnp.maximum(h + b1_ref[...], 0.0)
    z = jnp.sum(h * w2_ref[...], axis=1, keepdims=True) + b2_ref[...]
    o_ref[...] = jax.nn.sigmoid(z)


def _mlp(pooled, W1, b1, W2, b2):
    BT = 2048
    return pl.pallas_call(
        _mlp_body,
        grid=(BATCH // BT,),
        in_specs=[
            pl.BlockSpec((BT, EMBED), lambda i: (i, 0)),
            pl.BlockSpec((EMBED, HIDDEN), lambda i: (0, 0)),
            pl.BlockSpec((1, HIDDEN), lambda i: (0, 0)),
            pl.BlockSpec((1, HIDDEN), lambda i: (0, 0)),
            pl.BlockSpec((1, 1), lambda i: (0, 0)),
        ],
        out_specs=pl.BlockSpec((BT, OUT), lambda i: (i, 0)),
        out_shape=jax.ShapeDtypeStruct((BATCH, OUT), jnp.float32),
    )(pooled, W1, b1.reshape(1, HIDDEN), W2.reshape(1, HIDDEN),
      b2.reshape(1, OUT))


def kernel(inputs, emb_table, W1, b1, W2, b2):
    pooled = _pool(inputs.astype(jnp.int32), emb_table)
    return _mlp(pooled, W1, b1, W2, b2)


# speedup vs baseline: 10.2918x; 10.2918x over previous
import functools

import jax
import jax.numpy as jnp
from jax import lax
from jax.experimental import pallas as pl
from jax.experimental.pallas import tpu as pltpu
from jax.experimental.pallas import tpu_sc as plsc

VOCAB = 1000000
EMBED = 32
HIDDEN = 256
OUT = 1
BATCH = 16384
HIST = 200

NC = 2
NS = 16
NW = NC * NS
ELEMS = BATCH // NW
CB = 16
NCHUNK = ELEMS // CB
H0 = 96
H1 = HIST - H0


def _pool_body(inputs_hbm, table_hbm, out_hbm, idx_v, rows_v, pool_v, sem):
    c = lax.axis_index("c")
    s = lax.axis_index("s")
    wid = s * NC + c
    base = wid * ELEMS

    def chunk_body(ci, _):
        row0 = base + ci * CB
        pltpu.sync_copy(inputs_hbm.at[pl.ds(row0, CB)], idx_v)

        def elem_body(e, _):
            cp0 = pltpu.async_copy(
                table_hbm.at[idx_v.at[e, pl.ds(0, H0)]],
                rows_v.at[pl.ds(0, H0)], sem)
            cp1 = pltpu.async_copy(
                table_hbm.at[idx_v.at[e, pl.ds(H0, H1)]],
                rows_v.at[pl.ds(H0, H1)], sem)
            cp0.wait()
            cp1.wait()

            def row_body(l, accs):
                a0, a1 = accs
                return (a0 + rows_v[l, pl.ds(0, 16)],
                        a1 + rows_v[l, pl.ds(16, 16)])

            a0, a1 = lax.fori_loop(
                0, HIST, row_body,
                (jnp.zeros((16,), jnp.float32), jnp.zeros((16,), jnp.float32)),
                unroll=4)
            pool_v[e, pl.ds(0, 16)] = a0 * (1.0 / HIST)
            pool_v[e, pl.ds(16, 16)] = a1 * (1.0 / HIST)
            return 0

        lax.fori_loop(0, CB, elem_body, 0)
        pltpu.sync_copy(pool_v, out_hbm.at[pl.ds(row0, CB)])
        return 0

    lax.fori_loop(0, NCHUNK, chunk_body, 0)


def _pool(inputs, emb_table):
    mesh = plsc.VectorSubcoreMesh(core_axis_name="c", subcore_axis_name="s")
    fn = functools.partial(
        pl.kernel,
        mesh=mesh,
        compiler_params=pltpu.CompilerParams(use_tc_tiling_on_sc=False),
        out_type=jax.ShapeDtypeStruct((BATCH, EMBED), jnp.float32),
        scratch_types=[
            pltpu.VMEM((CB, HIST), jnp.int32),
            pltpu.VMEM((HIST, EMBED), jnp.float32),
            pltpu.VMEM((CB, EMBED), jnp.float32),
            pltpu.SemaphoreType.DMA,
        ],
    )(_pool_body)
    return fn(inputs, emb_table)


def _mlp_body(x_ref, w1_ref, b1_ref, w2_ref, b2_ref, o_ref):
    x = x_ref[...]
    h = jnp.dot(x, w1_ref[...], preferred_element_type=jnp.float32)
    h = jnp.maximum(h + b1_ref[...], 0.0)
    z = jnp.sum(h * w2_ref[...], axis=1, keepdims=True) + b2_ref[...]
    o_ref[...] = jax.nn.sigmoid(z)


def _mlp(pooled, W1, b1, W2, b2):
    BT = 2048
    return pl.pallas_call(
        _mlp_body,
        grid=(BATCH // BT,),
        in_specs=[
            pl.BlockSpec((BT, EMBED), lambda i: (i, 0)),
            pl.BlockSpec((EMBED, HIDDEN), lambda i: (0, 0)),
            pl.BlockSpec((1, HIDDEN), lambda i: (0, 0)),
            pl.BlockSpec((1, HIDDEN), lambda i: (0, 0)),
            pl.BlockSpec((1, 1), lambda i: (0, 0)),
        ],
        out_specs=pl.BlockSpec((BT, OUT), lambda i: (i, 0)),
        out_shape=jax.ShapeDtypeStruct((BATCH, OUT), jnp.float32),
    )(pooled, W1, b1.reshape(1, HIDDEN), W2.reshape(1, HIDDEN),
      b2.reshape(1, OUT))


def kernel(inputs, emb_table, W1, b1, W2, b2):
    pooled = _pool(inputs.astype(jnp.int32), emb_table)
    return _mlp(pooled, W1, b1, W2, b2)


# speedup vs baseline: 13.8083x; 1.3417x over previous
import functools

import jax
import jax.numpy as jnp
from jax import lax
from jax.experimental import pallas as pl
from jax.experimental.pallas import tpu as pltpu
from jax.experimental.pallas import tpu_sc as plsc

VOCAB = 1000000
EMBED = 32
HIDDEN = 256
OUT = 1
BATCH = 16384
HIST = 200

NC = 2
NS = 16
NW = NC * NS
ELEMS = BATCH // NW
CB = 16
NCHUNK = ELEMS // CB
H0 = 96
H1 = HIST - H0


def _pool_body(inputs_hbm, table_hbm, out_hbm, idx_v, rows0_v, rows1_v,
               pool_v, sem0, sem1):
    c = lax.axis_index("c")
    s = lax.axis_index("s")
    wid = s * NC + c
    base = wid * ELEMS

    pltpu.sync_copy(inputs_hbm.at[pl.ds(base, ELEMS)], idx_v)

    def fire(e, rows_v, sem):
        pltpu.async_copy(
            table_hbm.at[idx_v.at[e, pl.ds(0, H0)]],
            rows_v.at[pl.ds(0, H0)], sem)
        pltpu.async_copy(
            table_hbm.at[idx_v.at[e, pl.ds(H0, H1)]],
            rows_v.at[pl.ds(H0, H1)], sem)

    def drain(rows_v, sem):
        pltpu.make_async_copy(
            table_hbm.at[pl.ds(0, H0)], rows_v.at[pl.ds(0, H0)], sem).wait()
        pltpu.make_async_copy(
            table_hbm.at[pl.ds(0, H1)], rows_v.at[pl.ds(H0, H1)], sem).wait()

    def accum(e, rows_v):
        def row_body(l, accs):
            a0, a1 = accs
            return (a0 + rows_v[l, pl.ds(0, 16)],
                    a1 + rows_v[l, pl.ds(16, 16)])

        a0, a1 = lax.fori_loop(
            0, HIST, row_body,
            (jnp.zeros((16,), jnp.float32), jnp.zeros((16,), jnp.float32)),
            unroll=4)
        ep = lax.rem(e, CB)
        pool_v[ep, pl.ds(0, 16)] = a0 * (1.0 / HIST)
        pool_v[ep, pl.ds(16, 16)] = a1 * (1.0 / HIST)

    fire(0, rows0_v, sem0)

    def pair_body(e, _):
        fire(e + 1, rows1_v, sem1)
        drain(rows0_v, sem0)
        accum(e, rows0_v)

        @pl.when(e + 2 < ELEMS)
        def _():
            fire(e + 2, rows0_v, sem0)

        drain(rows1_v, sem1)
        accum(e + 1, rows1_v)

        @pl.when(lax.rem(e + 1, CB) == CB - 1)
        def _():
            pltpu.sync_copy(
                pool_v, out_hbm.at[pl.ds(base + (e + 1) - (CB - 1), CB)])
        return 0

    lax.fori_loop(0, ELEMS // 2, lambda i, cr: pair_body(i * 2, cr), 0)


def _pool(inputs, emb_table):
    mesh = plsc.VectorSubcoreMesh(core_axis_name="c", subcore_axis_name="s")
    fn = functools.partial(
        pl.kernel,
        mesh=mesh,
        compiler_params=pltpu.CompilerParams(use_tc_tiling_on_sc=False),
        out_type=jax.ShapeDtypeStruct((BATCH, EMBED), jnp.float32),
        scratch_types=[
            pltpu.VMEM((ELEMS, HIST), jnp.int32),
            pltpu.VMEM((HIST, EMBED), jnp.float32),
            pltpu.VMEM((HIST, EMBED), jnp.float32),
            pltpu.VMEM((CB, EMBED), jnp.float32),
            pltpu.SemaphoreType.DMA,
            pltpu.SemaphoreType.DMA,
        ],
    )(_pool_body)
    return fn(inputs, emb_table)


def _mlp_body(x_ref, w1_ref, b1_ref, w2_ref, b2_ref, o_ref):
    x = x_ref[...]
    h = jnp.dot(x, w1_ref[...], preferred_element_type=jnp.float32)
    h = jnp.maximum(h + b1_ref[...], 0.0)
    z = jnp.sum(h * w2_ref[...], axis=1, keepdims=True) + b2_ref[...]
    o_ref[...] = jax.nn.sigmoid(z)


def _mlp(pooled, W1, b1, W2, b2):
    BT = 2048
    return pl.pallas_call(
        _mlp_body,
        grid=(BATCH // BT,),
        in_specs=[
            pl.BlockSpec((BT, EMBED), lambda i: (i, 0)),
            pl.BlockSpec((EMBED, HIDDEN), lambda i: (0, 0)),
            pl.BlockSpec((1, HIDDEN), lambda i: (0, 0)),
            pl.BlockSpec((1, HIDDEN), lambda i: (0, 0)),
            pl.BlockSpec((1, 1), lambda i: (0, 0)),
        ],
        out_specs=pl.BlockSpec((BT, OUT), lambda i: (i, 0)),
        out_shape=jax.ShapeDtypeStruct((BATCH, OUT), jnp.float32),
    )(pooled, W1, b1.reshape(1, HIDDEN), W2.reshape(1, HIDDEN),
      b2.reshape(1, OUT))


def kernel(inputs, emb_table, W1, b1, W2, b2):
    pooled = _pool(inputs.astype(jnp.int32), emb_table)
    return _mlp(pooled, W1, b1, W2, b2)


# speedup vs baseline: 32.0055x; 2.3179x over previous
import functools

import jax
import jax.numpy as jnp
from jax import lax
from jax.experimental import pallas as pl
from jax.experimental.pallas import tpu as pltpu
from jax.experimental.pallas import tpu_sc as plsc

VOCAB = 1000000
EMBED = 32
HIDDEN = 256
OUT = 1
BATCH = 16384
HIST = 200

NC = 2
NS = 16
NW = NC * NS
ELEMS = BATCH // NW
CB = 16
H0 = 96
H1 = HIST - H0

PACK = 128 // EMBED
XROWS = BATCH // PACK
HP = PACK * HIDDEN


def _pool_body(idx_hbm, table_hbm, out_hbm, idxa_v, idxb_v, rows0_v,
               rows1_v, rows2_v, rows3_v, rows4_v, rows5_v, rows6_v,
               rows7_v, pool_v, sem0, sem1, sem2, sem3, sem4, sem5, sem6,
               sem7, isema, isemb):
    c = lax.axis_index("c")
    s = lax.axis_index("s")
    wid = s * NC + c
    base = wid * ELEMS
    EH = ELEMS // 4

    def accum(e, rows_v):
        def row_body(j, accs):
            c0, c1, c2, c3, c4, c5, c6, c7 = accs
            l = j * 4
            return (c0 + rows_v[l, pl.ds(0, 16)],
                    c1 + rows_v[l, pl.ds(16, 16)],
                    c2 + rows_v[l + 1, pl.ds(0, 16)],
                    c3 + rows_v[l + 1, pl.ds(16, 16)],
                    c4 + rows_v[l + 2, pl.ds(0, 16)],
                    c5 + rows_v[l + 2, pl.ds(16, 16)],
                    c6 + rows_v[l + 3, pl.ds(0, 16)],
                    c7 + rows_v[l + 3, pl.ds(16, 16)])

        z = jnp.zeros((16,), jnp.float32)
        c0, c1, c2, c3, c4, c5, c6, c7 = lax.fori_loop(
            0, HIST // 4, row_body, (z,) * 8, unroll=2)
        a0 = (c0 + c2) + (c4 + c6)
        a1 = (c1 + c3) + (c5 + c7)
        ep = lax.rem(e, CB)
        pool_v[pl.ds(ep * EMBED, 16)] = a0 * (1.0 / HIST)
        pool_v[pl.ds(ep * EMBED + 16, 16)] = a1 * (1.0 / HIST)

    bufs = (rows0_v, rows1_v, rows2_v, rows3_v,
            rows4_v, rows5_v, rows6_v, rows7_v)
    sems = (sem0, sem1, sem2, sem3, sem4, sem5, sem6, sem7)

    def stage(ch, idx_v, isem):
        pltpu.async_copy(
            idx_hbm.at[pl.ds((base + ch * EH) * HIST, EH * HIST)],
            idx_v, isem)

    def wait_stage(idx_v, isem):
        pltpu.make_async_copy(
            idx_hbm.at[pl.ds(0, EH * HIST)], idx_v, isem).wait()

    def run_chunk(ch, idx_v):
        def fire(e, rows_v, sem):
            pltpu.async_copy(
                table_hbm.at[idx_v.at[pl.ds(e * HIST, H0)]],
                rows_v.at[pl.ds(0, H0)], sem)
            pltpu.async_copy(
                table_hbm.at[idx_v.at[pl.ds(e * HIST + H0, H1)]],
                rows_v.at[pl.ds(H0, H1)], sem)

        def drain(rows_v, sem):
            pltpu.make_async_copy(
                table_hbm.at[pl.ds(0, H0)], rows_v.at[pl.ds(0, H0)],
                sem).wait()
            pltpu.make_async_copy(
                table_hbm.at[pl.ds(0, H1)], rows_v.at[pl.ds(H0, H1)],
                sem).wait()

        for k in range(8):
            fire(k, bufs[k], sems[k])

        def oct_body(e, _):
            eg = ch * EH + e
            for k in range(8):
                drain(bufs[k], sems[k])
                accum(eg + k, bufs[k])

                @pl.when(e + k + 8 < EH)
                def _(k=k):
                    fire(e + k + 8, bufs[k], sems[k])

            @pl.when(lax.rem(eg + 7, CB) == CB - 1)
            def _():
                pltpu.sync_copy(
                    pool_v,
                    out_hbm.at[pl.ds((base + (eg + 7) - (CB - 1)) * EMBED,
                                     CB * EMBED)])
            return 0

        lax.fori_loop(0, EH // 8, lambda i, cr: oct_body(i * 8, cr), 0)

    pltpu.sync_copy(idx_hbm.at[pl.ds(base * HIST, EH * HIST)], idxa_v)
    stage(1, idxb_v, isemb)
    run_chunk(0, idxa_v)
    stage(2, idxa_v, isema)
    wait_stage(idxb_v, isemb)
    run_chunk(1, idxb_v)
    stage(3, idxb_v, isemb)
    wait_stage(idxa_v, isema)
    run_chunk(2, idxa_v)
    wait_stage(idxb_v, isemb)
    run_chunk(3, idxb_v)


def _pool(idx_flat, emb_table):
    mesh = plsc.VectorSubcoreMesh(core_axis_name="c", subcore_axis_name="s")
    fn = functools.partial(
        pl.kernel,
        mesh=mesh,
        compiler_params=pltpu.CompilerParams(use_tc_tiling_on_sc=False),
        out_type=jax.ShapeDtypeStruct((BATCH * EMBED,), jnp.float32),
        scratch_types=[
            pltpu.VMEM((ELEMS * HIST // 4,), jnp.int32),
            pltpu.VMEM((ELEMS * HIST // 4,), jnp.int32),
            pltpu.VMEM((HIST, EMBED), jnp.float32),
            pltpu.VMEM((HIST, EMBED), jnp.float32),
            pltpu.VMEM((HIST, EMBED), jnp.float32),
            pltpu.VMEM((HIST, EMBED), jnp.float32),
            pltpu.VMEM((HIST, EMBED), jnp.float32),
            pltpu.VMEM((HIST, EMBED), jnp.float32),
            pltpu.VMEM((HIST, EMBED), jnp.float32),
            pltpu.VMEM((HIST, EMBED), jnp.float32),
            pltpu.VMEM((CB * EMBED,), jnp.float32),
            pltpu.SemaphoreType.DMA,
            pltpu.SemaphoreType.DMA,
            pltpu.SemaphoreType.DMA,
            pltpu.SemaphoreType.DMA,
            pltpu.SemaphoreType.DMA,
            pltpu.SemaphoreType.DMA,
            pltpu.SemaphoreType.DMA,
            pltpu.SemaphoreType.DMA,
            pltpu.SemaphoreType.DMA,
            pltpu.SemaphoreType.DMA,
        ],
    )(_pool_body)
    return fn(idx_flat, emb_table)


DB = 16384
DR = DB // PACK
DG = (VOCAB + DB - 1) // DB
VOCAB_PAD = DG * DB


def _detile_body(x_ref, e_ref, o_ref):
    x = x_ref[...]
    acc = None
    for a in range(PACK):
        p = jax.lax.dot_general(
            x[:, a * DR:(a + 1) * DR], e_ref[:, a * 128:(a + 1) * 128],
            (((0,), (0,)), ((), ())), preferred_element_type=jnp.float32)
        acc = p if acc is None else acc + p
    o_ref[...] = acc


def _detile(emb_t):
    eye = jnp.eye(EMBED, dtype=jnp.float32)
    sel = jnp.concatenate(
        [jnp.pad(eye, ((0, 0), (EMBED * a, 128 - EMBED * (a + 1))))
         for a in range(PACK)], axis=1)
    return pl.pallas_call(
        _detile_body,
        grid=(DG,),
        in_specs=[pl.BlockSpec((EMBED, DB), lambda i: (0, i)),
                  pl.BlockSpec((EMBED, PACK * 128), lambda i: (0, 0))],
        out_specs=pl.BlockSpec((DR, 128), lambda i: (i, 0)),
        out_shape=jax.ShapeDtypeStruct((VOCAB_PAD // PACK, 128),
                                       jnp.float32),
    )(emb_t, sel)


def _mlp_body(x_ref, w1_ref, b1_ref, w2_ref, s_ref, b2_ref, o_ref):
    x = x_ref[...]
    h = jnp.dot(x, w1_ref[...], preferred_element_type=jnp.float32)
    h = jnp.maximum(h + b1_ref[...], 0.0)
    t = h * w2_ref[...]
    z = jnp.dot(t, s_ref[...], preferred_element_type=jnp.float32)
    o_ref[...] = jax.nn.sigmoid(z + b2_ref[...])


def _mlp(xp, W1p, b1p, W2p, S, b2):
    BT = 512
    return pl.pallas_call(
        _mlp_body,
        grid=(XROWS // BT,),
        in_specs=[
            pl.BlockSpec((BT, 128), lambda i: (i, 0)),
            pl.BlockSpec((128, HP), lambda i: (0, 0)),
            pl.BlockSpec((1, HP), lambda i: (0, 0)),
            pl.BlockSpec((1, HP), lambda i: (0, 0)),
            pl.BlockSpec((HP, PACK), lambda i: (0, 0)),
            pl.BlockSpec((1, 1), lambda i: (0, 0)),
        ],
        out_specs=pl.BlockSpec((BT, PACK), lambda i: (i, 0)),
        out_shape=jax.ShapeDtypeStruct((XROWS, PACK), jnp.float32),
    )(xp, W1p, b1p, W2p, S, b2.reshape(1, OUT))


def kernel(inputs, emb_table, W1, b1, W2, b2):
    i32 = inputs.astype(jnp.int32)
    rs = DR.bit_length() - 1
    phi = ((i32 & ~(DB - 1)) | ((i32 & (DR - 1)) << 2)
           | ((i32 >> rs) & (PACK - 1)))
    idx_flat = phi.reshape(BATCH * HIST)
    emb_lin = _detile(emb_table.T).reshape(VOCAB_PAD, EMBED)
    pooled = _pool(idx_flat, emb_lin)
    xp = pooled.reshape(XROWS, 128)
    W1p = jax.scipy.linalg.block_diag(*([W1] * PACK))
    b1p = jnp.tile(b1, PACK).reshape(1, HP)
    W2p = jnp.tile(W2.reshape(HIDDEN), PACK).reshape(1, HP)
    S = jnp.repeat(jnp.eye(PACK, dtype=jnp.float32), HIDDEN, axis=0)
    out4 = _mlp(xp, W1p, b1p, W2p, S, b2)
    return out4.reshape(BATCH, OUT)
